# baseline (speedup 1.0000x reference)
import functools

import jax
import jax.numpy as jnp
from jax import lax
from jax.experimental import pallas as pl
from jax.experimental.pallas import tpu as pltpu
from jax.experimental.pallas import tpu_sc as plsc

B = 8
N = 10000
L = 16
C = 128
HID = 256
NOUT = 2500

NTILES = 32
CHUNK = 128
ROWS_PAD = 163840
RPT = ROWS_PAD // NTILES
NCH = RPT // CHUNK
NPV = ROWS_PAD * C // (L * C)

TB = 512
NB = NPV // TB


def _sc_gather(x, idx2d):
    mesh = plsc.VectorSubcoreMesh(core_axis_name="c", subcore_axis_name="s")

    @functools.partial(
        pl.kernel,
        mesh=mesh,
        out_type=jax.ShapeDtypeStruct((B, ROWS_PAD, C), jnp.float32),
        scratch_types=[
            pltpu.VMEM((NCH, CHUNK), jnp.int32),
            pltpu.VMEM((CHUNK, C), jnp.float32),
            pltpu.SemaphoreType.DMA,
        ],
    )
    def gather_kernel(x_hbm, idx_hbm, feat_hbm, idx_v, rows_v, gsem):
        wid = lax.axis_index("s") * 2 + lax.axis_index("c")
        pltpu.sync_copy(idx_hbm.at[pl.ds(wid * NCH, NCH)], idx_v)
        base = wid * RPT
        for b in range(B):
            def body(ci, carry):
                cp = pltpu.make_async_copy(
                    x_hbm.at[b].at[idx_v.at[ci]], rows_v, gsem)
                cp.start()
                cp.wait()
                pltpu.sync_copy(
                    rows_v, feat_hbm.at[b].at[pl.ds(base + ci * CHUNK, CHUNK)])
                return carry
            lax.fori_loop(0, NCH, body, 0)

    return gather_kernel(x, idx2d)


def _tc_dense(feat, down_transform, W, b2d):
    def body(feat_ref, t_ref, w_ref, b_ref, out_ref):
        nb = pl.program_id(0)
        bb = pl.program_id(1)
        h = jnp.dot(feat_ref[0], w_ref[...], preferred_element_type=jnp.float32)
        h = jnp.maximum(h + b_ref[...], 0.0)

        @pl.when(nb == 0)
        def _():
            out_ref[bb] = jnp.zeros_like(out_ref[bb])

        out_ref[bb] += jnp.dot(t_ref[...], h, preferred_element_type=jnp.float32)

    return pl.pallas_call(
        body,
        grid=(NB, B),
        in_specs=[
            pl.BlockSpec((1, TB, L * C), lambda nb, bb: (bb, nb, 0)),
            pl.BlockSpec((NOUT, TB), lambda nb, bb: (0, nb)),
            pl.BlockSpec((L * C, HID), lambda nb, bb: (0, 0)),
            pl.BlockSpec((1, HID), lambda nb, bb: (0, 0)),
        ],
        out_specs=pl.BlockSpec((B, NOUT, HID), lambda nb, bb: (0, 0, 0)),
        out_shape=jax.ShapeDtypeStruct((B, NOUT, HID), jnp.float32),
        compiler_params=pltpu.CompilerParams(
            dimension_semantics=("arbitrary", "arbitrary"),
        ),
    )(feat, down_transform, W, b2d)


def kernel(x, indices, down_transform, W, b):
    flat_idx = indices.reshape(-1).astype(jnp.int32)
    flat_idx = jnp.pad(flat_idx, (0, ROWS_PAD - N * L))
    idx2d = flat_idx.reshape(NTILES * NCH, CHUNK)
    feat = _sc_gather(x, idx2d)
    feat = feat.reshape(B, NPV, L * C)
    t_pad = jnp.pad(down_transform, ((0, 0), (0, NPV - N)))
    out = _tc_dense(feat, t_pad, W, b.reshape(1, HID))
    return out


# speedup vs baseline: 1.6747x; 1.6747x over previous
import functools

import jax
import jax.numpy as jnp
from jax import lax
from jax.experimental import pallas as pl
from jax.experimental.pallas import tpu as pltpu
from jax.experimental.pallas import tpu_sc as plsc

B = 8
N = 10000
L = 16
C = 128
HID = 256
NOUT = 2500

NTILES = 32
CHUNK = 64
ROWS_PAD = 163840
RPT = ROWS_PAD // NTILES
NCH = RPT // CHUNK
NPV = ROWS_PAD * C // (L * C)

TB = 512
NB = NPV // TB


def _sc_gather(x, idx2d):
    mesh = plsc.VectorSubcoreMesh(core_axis_name="c", subcore_axis_name="s")

    @functools.partial(
        pl.kernel,
        mesh=mesh,
        out_type=jax.ShapeDtypeStruct((B, ROWS_PAD, C), jnp.float32),
        scratch_types=[
            pltpu.VMEM((NCH, CHUNK), jnp.int32),
            pltpu.VMEM((B, CHUNK, C), jnp.float32),
            pltpu.SemaphoreType.DMA((B,)),
            pltpu.SemaphoreType.DMA((B,)),
        ],
    )
    def gather_kernel(x_hbm, idx_hbm, feat_hbm, idx_v, rows_v, gsem, ssem):
        wid = lax.axis_index("s") * 2 + lax.axis_index("c")
        pltpu.sync_copy(idx_hbm.at[pl.ds(wid * NCH, NCH)], idx_v)
        base = wid * RPT

        def gath(b, ci):
            return pltpu.make_async_copy(
                x_hbm.at[b].at[idx_v.at[ci]], rows_v.at[b], gsem.at[b])

        def scat(b, ci):
            return pltpu.make_async_copy(
                rows_v.at[b],
                feat_hbm.at[b].at[pl.ds(base + ci * CHUNK, CHUNK)],
                ssem.at[b])

        def body(ci, carry):
            for b in range(B):
                @pl.when(ci > 0)
                def _(b=b):
                    scat(b, ci - 1).wait()
                gath(b, ci).start()
            for b in range(B):
                gath(b, ci).wait()
                scat(b, ci).start()
            return carry

        lax.fori_loop(0, NCH, body, 0)
        for b in range(B):
            scat(b, NCH - 1).wait()

    return gather_kernel(x, idx2d)


def _tc_dense(feat, down_transform, W, b2d):
    def body(feat_ref, t_ref, w_ref, b_ref, out_ref):
        nb = pl.program_id(0)
        bb = pl.program_id(1)
        h = jnp.dot(feat_ref[0], w_ref[...], preferred_element_type=jnp.float32)
        h = jnp.maximum(h + b_ref[...], 0.0)

        @pl.when(nb == 0)
        def _():
            out_ref[bb] = jnp.zeros_like(out_ref[bb])

        out_ref[bb] += jnp.dot(t_ref[...], h, preferred_element_type=jnp.float32)

    return pl.pallas_call(
        body,
        grid=(NB, B),
        in_specs=[
            pl.BlockSpec((1, TB, L * C), lambda nb, bb: (bb, nb, 0)),
            pl.BlockSpec((NOUT, TB), lambda nb, bb: (0, nb)),
            pl.BlockSpec((L * C, HID), lambda nb, bb: (0, 0)),
            pl.BlockSpec((1, HID), lambda nb, bb: (0, 0)),
        ],
        out_specs=pl.BlockSpec((B, NOUT, HID), lambda nb, bb: (0, 0, 0)),
        out_shape=jax.ShapeDtypeStruct((B, NOUT, HID), jnp.float32),
        compiler_params=pltpu.CompilerParams(
            dimension_semantics=("arbitrary", "arbitrary"),
        ),
    )(feat, down_transform, W, b2d)


def kernel(x, indices, down_transform, W, b):
    flat_idx = indices.reshape(-1).astype(jnp.int32)
    flat_idx = jnp.pad(flat_idx, (0, ROWS_PAD - N * L))
    idx2d = flat_idx.reshape(NTILES * NCH, CHUNK)
    feat = _sc_gather(x, idx2d)
    feat = feat.reshape(B, NPV, L * C)
    t_pad = jnp.pad(down_transform, ((0, 0), (0, NPV - N)))
    out = _tc_dense(feat, t_pad, W, b.reshape(1, HID))
    return out
